# baseline (speedup 1.0000x reference)
import jax
import jax.numpy as jnp
from jax import lax
from jax.experimental import pallas as pl
from jax.experimental.pallas import tpu as pltpu

_TAPER = (128, 128, 256, 512)
_BT = 1024
_NS = 3


def _softmax_rows(logits):
    m = jnp.max(logits, axis=-1, keepdims=True)
    e = jnp.exp(logits - m)
    return e * (1.0 / jnp.sum(e, axis=-1, keepdims=True))


def _router_body(x_ref, w_ref, b_ref, o_ref, xbuf, ybuf, in_sem, out_sem):
    T = x_ref.shape[0]
    nsteps = 3 + T // _BT
    dn = (((1,), (1,)), ((), ()))

    def nrows(k_static):
        return _TAPER[k_static] if k_static < 4 else _BT

    def static_off(k_static):
        if k_static < 4:
            return sum(_TAPER[:k_static])
        return (k_static - 3) * _BT

    def in_copy(off, slot, n):
        return pltpu.make_async_copy(
            x_ref.at[pl.ds(off, n), :],
            xbuf.at[slot, pl.ds(0, n), :],
            in_sem.at[slot])

    def out_copy(off, slot, n):
        return pltpu.make_async_copy(
            ybuf.at[slot, pl.ds(0, n), :],
            o_ref.at[pl.ds(off, n), :],
            out_sem.at[slot])

    w = w_ref[...]
    bias = b_ref[...]

    def compute(off, slot, n):
        logits = lax.dot_general(
            xbuf[slot, :n, :], w, dn, preferred_element_type=jnp.float32) + bias
        ybuf[slot, :n, :] = _softmax_rows(logits)
        out_copy(off, slot, n).start()

    for k in range(_NS):
        in_copy(static_off(k), k, nrows(k)).start()

    for k in range(7):
        slot = k % _NS
        n = nrows(k)
        in_copy(static_off(k), slot, n).wait()
        if k >= _NS:
            out_copy(static_off(k - _NS), slot, nrows(k - _NS)).wait()
        compute(static_off(k), slot, n)
        nxt = k + _NS
        if nxt < nsteps:
            in_copy(static_off(nxt), nxt % _NS, nrows(nxt)).start()

    ngroups = (nsteps - 7) // _NS
    nleft = (nsteps - 7) % _NS

    def group(g, carry):
        for j in range(_NS):
            slot = (7 + j) % _NS
            k = 7 + g * _NS + j
            off = (k - 3) * _BT
            in_copy(off, slot, _BT).wait()
            out_copy(off - _NS * _BT, slot, _BT).wait()
            compute(off, slot, _BT)

            @pl.when(k + _NS < nsteps)
            def _():
                in_copy(off + _NS * _BT, slot, _BT).start()
        return carry

    lax.fori_loop(0, ngroups, group, 0)

    for j in range(nleft):
        k = 7 + ngroups * _NS + j
        slot = k % _NS
        off = (k - 3) * _BT
        in_copy(off, slot, _BT).wait()
        out_copy(off - _NS * _BT, slot, _BT).wait()
        compute(off, slot, _BT)

    for k in range(nsteps - _NS, nsteps):
        out_copy((k - 3) * _BT, k % _NS, _BT).wait()


def kernel(x, W, b):
    T, D = x.shape
    E = W.shape[0]
    return pl.pallas_call(
        _router_body,
        in_specs=[
            pl.BlockSpec(memory_space=pl.ANY),
            pl.BlockSpec((E, D), lambda: (0, 0)),
            pl.BlockSpec((1, E), lambda: (0, 0)),
        ],
        out_specs=pl.BlockSpec(memory_space=pl.ANY),
        out_shape=jax.ShapeDtypeStruct((T, E), jnp.float32),
        scratch_shapes=[
            pltpu.VMEM((_NS, _BT, D), jnp.float32),
            pltpu.VMEM((_NS, _BT, E), jnp.float32),
            pltpu.SemaphoreType.DMA((_NS,)),
            pltpu.SemaphoreType.DMA((_NS,)),
        ],
    )(x, W, b.reshape(1, E))


# speedup vs baseline: 1.0201x; 1.0201x over previous
import jax
import jax.numpy as jnp
from jax import lax
from jax.experimental import pallas as pl
from jax.experimental.pallas import tpu as pltpu


def _router_body(x_ref, w_ref, b_ref, o_ref):
    logits = lax.dot_general(
        x_ref[...], w_ref[...],
        dimension_numbers=(((1,), (1,)), ((), ())),
        preferred_element_type=jnp.float32,
    )
    logits = logits + b_ref[...]
    m = jnp.max(logits, axis=-1, keepdims=True)
    e = jnp.exp(logits - m)
    o_ref[...] = e * (1.0 / jnp.sum(e, axis=-1, keepdims=True))


def kernel(x, W, b):
    T, D = x.shape
    E = W.shape[0]
    BT = 1024
    return pl.pallas_call(
        _router_body,
        grid=(T // BT,),
        in_specs=[
            pl.BlockSpec((BT, D), lambda i: (i, 0)),
            pl.BlockSpec((E, D), lambda i: (0, 0)),
            pl.BlockSpec((1, E), lambda i: (0, 0)),
        ],
        out_specs=pl.BlockSpec((BT, E), lambda i: (i, 0)),
        out_shape=jax.ShapeDtypeStruct((T, E), jnp.float32),
        compiler_params=pltpu.CompilerParams(
            dimension_semantics=("arbitrary",),
        ),
    )(x, W, b.reshape(1, E))
